# Initial kernel scaffold; baseline (speedup 1.0000x reference)
import jax
import jax.numpy as jnp
from jax.experimental import pallas as pl


def kernel(fm_p2, fm_p3, fm_p4, fm_p5, rois, valid_mask, roi_size_pred):
    raise NotImplementedError("write your pallas kernel here")


# speedup vs baseline: 21.4621x; 21.4621x over previous
import functools

import jax
import jax.numpy as jnp
from jax import lax
from jax.experimental import pallas as pl
from jax.experimental.pallas import tpu as pltpu
from jax.experimental.pallas import tpu_sc as plsc

OUT = 7
C = 256
NLANE = 16
NWORKERS = 32


def _dg(v, idx):
    return lax.gather(
        v,
        idx[:, None],
        dimension_numbers=lax.GatherDimensionNumbers(
            offset_dims=(), collapsed_slice_dims=(0,), start_index_map=(0,)
        ),
        slice_sizes=(1,),
        mode=lax.GatherScatterMode.PROMISE_IN_BOUNDS,
    )


def _splat(x):
    return jnp.full((NLANE,), x, dtype=jnp.int32)


def _make_run(n_rois, rows_total):
    rpw = n_rois // NWORKERS
    mesh = plsc.VectorSubcoreMesh(core_axis_name="c", subcore_axis_name="s")

    @functools.partial(
        pl.kernel,
        mesh=mesh,
        out_type=jax.ShapeDtypeStruct((n_rois, OUT, OUT, C), jnp.float32),
        scratch_types=[
            pltpu.VMEM((rpw, NLANE), jnp.float32),
            pltpu.VMEM((rpw, NLANE), jnp.int32),
            pltpu.VMEM((OUT, 2, NLANE, C), jnp.float32),
            pltpu.VMEM((OUT, OUT, C), jnp.float32),
            pltpu.SemaphoreType.DMA,
        ],
    )
    def run(table_h, pf_h, pi_h, out_h, pf_v, pi_v, gbuf, obuf, gsem):
        wid = lax.axis_index("s") * 2 + lax.axis_index("c")
        roi0 = wid * rpw
        pltpu.sync_copy(pf_h.at[pl.ds(roi0, rpw)], pf_v)
        pltpu.sync_copy(pi_h.at[pl.ds(roi0, rpw)], pi_v)

        iota = lax.iota(jnp.int32, NLANE)
        iota_f = iota.astype(jnp.float32)
        shift7 = jnp.maximum(iota - 7, 0)

        def roi_body(r, carry):
            pfv = pf_v[r]
            piv = pi_v[r]
            ay = pfv[0]
            dy = pfv[1]
            ax = pfv[2]
            dx = pfv[3]
            hm1f = pfv[4]
            base = piv[0]
            fw = piv[1]
            hm1 = piv[2]

            ys = ay + iota_f * dy
            my = jnp.where((ys >= 0.0) & (ys <= hm1f), 1.0, 0.0)
            ysc = jnp.clip(ys, 0.0, hm1f)
            y0 = ysc.astype(jnp.int32)
            wy = ysc - y0.astype(jnp.float32)
            yp = jnp.minimum(y0 + 1, hm1)

            xs = ax + iota_f * dx
            mx = jnp.where((xs >= 0.0) & (xs <= hm1f), 1.0, 0.0)
            xsc = jnp.clip(xs, 0.0, hm1f)
            x0 = xsc.astype(jnp.int32)
            wx = xsc - x0.astype(jnp.float32)
            xp = jnp.minimum(x0 + 1, hm1)

            xcomb = jnp.where(iota < 7, x0, _dg(xp, shift7))
            rb0 = base + y0 * fw
            rb1 = base + yp * fw

            copies = []
            for i in range(OUT):
                si = _splat(i)
                idx0 = _dg(rb0, si) + xcomb
                idx1 = _dg(rb1, si) + xcomb
                copies.append(pltpu.async_copy(table_h.at[idx0], gbuf.at[i, 0], gsem))
                copies.append(pltpu.async_copy(table_h.at[idx1], gbuf.at[i, 1], gsem))
            for cp in copies:
                cp.wait()

            def row_body(i, carry_i):
                sI = _splat(i)
                wyi = _dg(wy, sI)
                myi = _dg(my, sI)
                w0y = (1.0 - wyi) * myi
                w1y = wyi * myi
                w00v = (1.0 - wx) * mx * w0y
                w01v = wx * mx * w0y
                w10v = (1.0 - wx) * mx * w1y
                w11v = wx * mx * w1y

                def col_body(j, carry_j):
                    sj = _splat(j)
                    c00 = _dg(w00v, sj)
                    c01 = _dg(w01v, sj)
                    c10 = _dg(w10v, sj)
                    c11 = _dg(w11v, sj)
                    for ch in range(C // NLANE):
                        sl = pl.ds(ch * NLANE, NLANE)
                        va = gbuf[i, 0, j, sl]
                        vb = gbuf[i, 0, j + 7, sl]
                        vc = gbuf[i, 1, j, sl]
                        vd = gbuf[i, 1, j + 7, sl]
                        obuf[i, j, sl] = va * c00 + vb * c01 + vc * c10 + vd * c11
                    return carry_j

                lax.fori_loop(0, OUT, col_body, 0)
                return carry_i

            lax.fori_loop(0, OUT, row_body, 0)
            pltpu.sync_copy(obuf, out_h.at[roi0 + r])
            return carry

        lax.fori_loop(0, rpw, roi_body, 0)

    return run


def kernel(fm_p2, fm_p3, fm_p4, fm_p5, rois, valid_mask, roi_size_pred):
    del valid_mask, roi_size_pred
    B, N = rois.shape[0], rois.shape[1]
    fms = (fm_p2, fm_p3, fm_p4, fm_p5)
    sizes = [fm.shape[1] * fm.shape[2] for fm in fms]
    rows_per_b = sum(sizes)
    offs = [0]
    for s in sizes[:-1]:
        offs.append(offs[-1] + s)

    table = jnp.concatenate(
        [fm.reshape(B, -1, C) for fm in fms], axis=1
    ).reshape(-1, C)

    x1 = rois[..., 0]
    y1 = rois[..., 1]
    x2 = rois[..., 2]
    y2 = rois[..., 3]
    area = (x2 - x1) * (y2 - y1)
    lv = jnp.log(jnp.sqrt(jnp.maximum(area, 1e-12)) / 224.0) / jnp.log(2.0) + 4.0
    lv = jnp.clip(jnp.round(lv).astype(jnp.int32), 2, 5)

    stride = jnp.exp2(lv.astype(jnp.float32))
    fmhw = fm_p2.shape[1] * 4.0 / stride
    hm1f = fmhw - 1.0
    y1n = (y1 * (1.0 / stride)) / fmhw
    y2n = (y2 * (1.0 / stride)) / fmhw
    x1n = (x1 * (1.0 / stride)) / fmhw
    x2n = (x2 * (1.0 / stride)) / fmhw
    ay = y1n * hm1f
    dy = (y2n - y1n) * hm1f / (OUT - 1.0)
    ax = x1n * hm1f
    dx = (x2n - x1n) * hm1f / (OUT - 1.0)

    fw = fmhw.astype(jnp.int32)
    hm1i = fw - 1
    boff = jnp.arange(B, dtype=jnp.int32)[:, None] * rows_per_b
    base = jnp.asarray(offs, dtype=jnp.int32)[lv - 2] + boff

    zf = jnp.zeros_like(ay)
    zi = jnp.zeros_like(fw)
    pf = jnp.stack([ay, dy, ax, dx, hm1f] + [zf] * 11, axis=-1).reshape(B * N, NLANE)
    pi = jnp.stack([base, fw, hm1i] + [zi] * 13, axis=-1).reshape(B * N, NLANE)

    run = _make_run(B * N, table.shape[0])
    out = run(table, pf, pi)
    return out.reshape(B, N, OUT, OUT, C).astype(jnp.float16)
